# baseline (speedup 1.0000x reference)
import functools

import jax
import jax.numpy as jnp
from jax import lax
from jax.experimental import pallas as pl
from jax.experimental.pallas import tpu as pltpu
from jax.experimental.pallas import tpu_sc as plsc

B, M, N, DA, DS = 16, 4096, 1024, 128, 128
R = B * M
T = B * N
NC, NS = 2, 16
NW = NC * NS
CT = 64
PBLK = 4096


def _proj_body(a_ref, w_ref, x_ref):
    x_ref[...] = jnp.maximum(
        lax.dot_general(a_ref[...], w_ref[...], (((1,), (1,)), ((), ())),
                        preferred_element_type=jnp.float32),
        0.0)


def _make_sc_body(nb):
    tpw = nb * N // NW
    qpb = NW // nb
    nch = tpw // CT
    nkc = N // 16 // qpb
    rmax = nb * M - 1

    def body(x_hbm, lens_hbm, out_hbm,
             lens_v, idx_v, w_v, bufA, bufB, outA, outB,
             semA, semB, semOA, semOB):
        wid = lax.axis_index("s") * NC + lax.axis_index("c")
        b = wid // qpb
        qh = wid % qpb
        pltpu.sync_copy(lens_hbm.at[b], lens_v)

        qsum = []
        for i in range(qpb):
            acc = lens_v[pl.ds(i * nkc * 16, 16)].astype(jnp.float32)
            for k in range(1, nkc):
                acc = acc + lens_v[pl.ds((i * nkc + k) * 16, 16)].astype(
                    jnp.float32)
            qsum.append(plsc.cumsum(acc)[15])
        carry = jnp.float32(0.0)
        for i in range(qpb - 1):
            carry = carry + jnp.where(qh >= i + 1, qsum[i], 0.0)

        boff = b * M
        base = qh * tpw
        for k in range(nkc):
            v = lens_v[pl.ds(base + k * 16, 16)]
            vf = v.astype(jnp.float32)
            ends = plsc.cumsum(vf) + carry
            starts = (ends - vf).astype(jnp.int32)
            carry = ends[15]
            gi = starts + boff
            inv = 1.0 / jnp.maximum(vf, 1.0)
            for j in range(4):
                idx_v[j, pl.ds(k * 16, 16)] = jnp.minimum(gi + j, rmax)
                w_v[j, pl.ds(k * 16, 16)] = jnp.where(v > j, inv, 0.0)

        bufs = (bufA, bufB)
        sems = (semA, semB)
        outs = (outA, outB)
        osems = (semOA, semOB)

        def fire(chunk, nbuf):
            buf, sem = bufs[nbuf], sems[nbuf]
            return [pltpu.async_copy(
                x_hbm.at[idx_v.at[j, pl.ds(chunk * CT, CT)]],
                buf.at[j], sem) for j in range(4)]

        def compute(chunk, nbuf):
            buf = bufs[nbuf]
            ov = outs[nbuf]

            @plsc.parallel_loop(0, CT // 16, unroll=1)
            def group_body(tg):
                wv0 = w_v[0, pl.ds(chunk * CT + tg * 16, 16)]
                wv1 = w_v[1, pl.ds(chunk * CT + tg * 16, 16)]
                wv2 = w_v[2, pl.ds(chunk * CT + tg * 16, 16)]
                wv3 = w_v[3, pl.ds(chunk * CT + tg * 16, 16)]
                for u in range(16):
                    t = tg * 16 + u
                    w0, w1, w2, w3 = wv0[u], wv1[u], wv2[u], wv3[u]
                    for g in range(8):
                        s = pl.ds(g * 16, 16)
                        ov[t, s] = (buf[0, t, s] * w0 + buf[1, t, s] * w1
                                    + buf[2, t, s] * w2 + buf[3, t, s] * w3)

        handles = [None, None]
        ohandles = [None, None]
        handles[0] = fire(0, 0)
        for chunk in range(nch):
            nbf = chunk % 2
            if chunk + 1 < nch:
                handles[1 - nbf] = fire(chunk + 1, 1 - nbf)
            for hdl in handles[nbf]:
                hdl.wait()
            if ohandles[nbf] is not None:
                ohandles[nbf].wait()
            compute(chunk, nbf)
            ohandles[nbf] = pltpu.async_copy(
                outs[nbf], out_hbm.at[pl.ds(wid * tpw + chunk * CT, CT)],
                osems[nbf])
        for oh in ohandles:
            if oh is not None:
                oh.wait()

    pool = pl.kernel(
        body,
        out_type=jax.ShapeDtypeStruct((nb * N, DS), jnp.float32),
        mesh=plsc.VectorSubcoreMesh(core_axis_name="c", subcore_axis_name="s"),
        compiler_params=pltpu.CompilerParams(needs_layout_passes=False),
        scratch_types=[
            pltpu.VMEM((N,), jnp.int32),
            pltpu.VMEM((4, tpw), jnp.int32),
            pltpu.VMEM((4, tpw), jnp.float32),
            pltpu.VMEM((4, CT, DS), jnp.float32),
            pltpu.VMEM((4, CT, DS), jnp.float32),
            pltpu.VMEM((CT, DS), jnp.float32),
            pltpu.VMEM((CT, DS), jnp.float32),
            pltpu.SemaphoreType.DMA,
            pltpu.SemaphoreType.DMA,
            pltpu.SemaphoreType.DMA,
            pltpu.SemaphoreType.DMA,
        ],
    )
    return pool


def _proj(a2):
    r = a2.shape[0]
    return pl.pallas_call(
        _proj_body,
        grid=(r // PBLK,),
        in_specs=[pl.BlockSpec((PBLK, DA), lambda i: (i, 0)),
                  pl.BlockSpec((DS, DA), lambda i: (0, 0))],
        out_specs=pl.BlockSpec((PBLK, DS), lambda i: (i, 0)),
        out_shape=jax.ShapeDtypeStruct((r, DS), jnp.float32),
    )


def kernel(atom_feats, atom_mask, molecule_atom_lens, W):
    del atom_mask
    hb = B // 2
    pool = _make_sc_body(hb)
    a2 = atom_feats.reshape(R, DA)
    halves = []
    for i in range(2):
        ah = a2[i * (R // 2):(i + 1) * (R // 2)]
        xh = _proj(ah)(ah, W)
        lh = molecule_atom_lens[i * hb:(i + 1) * hb]
        halves.append(pool(xh, lh))
    out = jnp.concatenate(halves, axis=0)
    return out.reshape(B, N, DS)


# speedup vs baseline: 1.3982x; 1.3982x over previous
import functools

import jax
import jax.numpy as jnp
from jax import lax
from jax.experimental import pallas as pl
from jax.experimental.pallas import tpu as pltpu
from jax.experimental.pallas import tpu_sc as plsc

B, M, N, DA, DS = 16, 4096, 1024, 128, 128
R = B * M
T = B * N
NC, NS = 2, 16
NW = NC * NS
CT = 64
PBLK = 4096


def _proj_body(a_ref, w_ref, x_ref):
    x_ref[...] = jnp.maximum(
        lax.dot_general(a_ref[...], w_ref[...], (((1,), (1,)), ((), ())),
                        preferred_element_type=jnp.float32),
        0.0)


def _make_sc_body(nb):
    tpw = nb * N // NW
    qpb = NW // nb
    nch = tpw // CT
    nkc = N // 16 // qpb
    rmax = nb * M - 1

    def body(x_hbm, lens_hbm, out_hbm,
             lens_v, idx_v, w_v, bufA, bufB, outA, outB,
             semA, semB, semOA, semOB):
        wid = lax.axis_index("s") * NC + lax.axis_index("c")
        b = wid // qpb
        qh = wid % qpb
        pltpu.sync_copy(lens_hbm.at[b], lens_v)

        qsum = []
        for i in range(qpb):
            acc = lens_v[pl.ds(i * nkc * 16, 16)].astype(jnp.float32)
            for k in range(1, nkc):
                acc = acc + lens_v[pl.ds((i * nkc + k) * 16, 16)].astype(
                    jnp.float32)
            qsum.append(plsc.cumsum(acc)[15])
        carry = jnp.float32(0.0)
        for i in range(qpb - 1):
            carry = carry + jnp.where(qh >= i + 1, qsum[i], 0.0)

        boff = b * M
        base = qh * tpw
        for k in range(nkc):
            v = lens_v[pl.ds(base + k * 16, 16)]
            vf = v.astype(jnp.float32)
            ends = plsc.cumsum(vf) + carry
            starts = (ends - vf).astype(jnp.int32)
            carry = ends[15]
            gi = starts + boff
            inv = 1.0 / jnp.maximum(vf, 1.0)
            for j in range(4):
                idx_v[j, pl.ds(k * 16, 16)] = jnp.minimum(gi + j, rmax)
                w_v[j, pl.ds(k * 16, 16)] = jnp.where(v > j, inv, 0.0)

        bufs = (bufA, bufB)
        sems = (semA, semB)
        outs = (outA, outB)
        osems = (semOA, semOB)

        def fire(chunk, nbuf):
            buf, sem = bufs[nbuf], sems[nbuf]
            return [pltpu.async_copy(
                x_hbm.at[idx_v.at[j, pl.ds(chunk * CT, CT)]],
                buf.at[j], sem) for j in range(4)]

        def compute(chunk, nbuf):
            buf = bufs[nbuf]
            ov = outs[nbuf]

            @plsc.parallel_loop(0, CT // 16, unroll=1)
            def group_body(tg):
                wv0 = w_v[0, pl.ds(chunk * CT + tg * 16, 16)]
                wv1 = w_v[1, pl.ds(chunk * CT + tg * 16, 16)]
                wv2 = w_v[2, pl.ds(chunk * CT + tg * 16, 16)]
                wv3 = w_v[3, pl.ds(chunk * CT + tg * 16, 16)]
                for u in range(16):
                    t = tg * 16 + u
                    w0, w1, w2, w3 = wv0[u], wv1[u], wv2[u], wv3[u]
                    for g in range(8):
                        s = pl.ds(g * 16, 16)
                        ov[t, s] = (buf[0, t, s] * w0 + buf[1, t, s] * w1
                                    + buf[2, t, s] * w2 + buf[3, t, s] * w3)

        handles = [None, None]
        ohandles = [None, None]
        handles[0] = fire(0, 0)
        for chunk in range(nch):
            nbf = chunk % 2
            if chunk + 1 < nch:
                handles[1 - nbf] = fire(chunk + 1, 1 - nbf)
            for hdl in handles[nbf]:
                hdl.wait()
            if ohandles[nbf] is not None:
                ohandles[nbf].wait()
            compute(chunk, nbf)
            ohandles[nbf] = pltpu.async_copy(
                outs[nbf], out_hbm.at[pl.ds(wid * tpw + chunk * CT, CT)],
                osems[nbf])
        for oh in ohandles:
            if oh is not None:
                oh.wait()

    pool = pl.kernel(
        body,
        out_type=jax.ShapeDtypeStruct((nb * N, DS), jnp.float32),
        mesh=plsc.VectorSubcoreMesh(core_axis_name="c", subcore_axis_name="s"),
        compiler_params=pltpu.CompilerParams(needs_layout_passes=False),
        scratch_types=[
            pltpu.VMEM((N,), jnp.int32),
            pltpu.VMEM((4, tpw), jnp.int32),
            pltpu.VMEM((4, tpw), jnp.float32),
            pltpu.VMEM((4, CT, DS), jnp.float32),
            pltpu.VMEM((4, CT, DS), jnp.float32),
            pltpu.VMEM((CT, DS), jnp.float32),
            pltpu.VMEM((CT, DS), jnp.float32),
            pltpu.SemaphoreType.DMA,
            pltpu.SemaphoreType.DMA,
            pltpu.SemaphoreType.DMA,
            pltpu.SemaphoreType.DMA,
        ],
    )
    return pool


def _proj(a2):
    r = a2.shape[0]
    return pl.pallas_call(
        _proj_body,
        grid=(r // PBLK,),
        in_specs=[pl.BlockSpec((PBLK, DA), lambda i: (i, 0)),
                  pl.BlockSpec((DS, DA), lambda i: (0, 0))],
        out_specs=pl.BlockSpec((PBLK, DS), lambda i: (i, 0)),
        out_shape=jax.ShapeDtypeStruct((r, DS), jnp.float32),
    )


def kernel(atom_feats, atom_mask, molecule_atom_lens, W):
    del atom_mask
    pool = _make_sc_body(B)
    a2 = atom_feats.reshape(R, DA)
    x = _proj(a2)(a2, W)
    out = pool(x, molecule_atom_lens)
    return out.reshape(B, N, DS)
